# Initial kernel scaffold; baseline (speedup 1.0000x reference)
import jax
import jax.numpy as jnp
from jax.experimental import pallas as pl


def kernel(logits, u):
    raise NotImplementedError("write your pallas kernel here")


# speedup vs baseline: 320.4366x; 320.4366x over previous
import jax
import jax.numpy as jnp
from jax.experimental import pallas as pl


_BLOCK_ROWS = 256


def _mask_kernel(l_ref, u_ref, o_ref):
    z = l_ref[...] - jnp.log(-jnp.log(u_ref[...]))
    m = jnp.max(z, axis=-1, keepdims=True)
    iota = jax.lax.broadcasted_iota(jnp.int32, z.shape, 1)
    fd = z.shape[-1]
    cand = jnp.where(z == m, iota, fd)
    idx = jnp.min(cand, axis=-1, keepdims=True)
    o_ref[...] = (iota == idx).astype(jnp.float32)


def kernel(logits, u):
    b, s, fd = logits.shape
    rows = b * s
    l2 = logits.reshape(rows, fd)
    u2 = u.reshape(rows, fd)
    out = pl.pallas_call(
        _mask_kernel,
        grid=(rows // _BLOCK_ROWS,),
        in_specs=[
            pl.BlockSpec((_BLOCK_ROWS, fd), lambda i: (i, 0)),
            pl.BlockSpec((_BLOCK_ROWS, fd), lambda i: (i, 0)),
        ],
        out_specs=pl.BlockSpec((_BLOCK_ROWS, fd), lambda i: (i, 0)),
        out_shape=jax.ShapeDtypeStruct((rows, fd), jnp.float32),
    )(l2, u2)
    return out.reshape(b, s, fd)


# speedup vs baseline: 406.5088x; 1.2686x over previous
import jax
import jax.numpy as jnp
from jax.experimental import pallas as pl
from jax.experimental.pallas import tpu as pltpu


_BLOCK_ROWS = 512


def _mask_kernel(l_ref, u_ref, o_ref):
    z = l_ref[...] - jnp.log(-jnp.log(u_ref[...]))
    m = jnp.max(z, axis=-1, keepdims=True)
    iota = jax.lax.broadcasted_iota(jnp.int32, z.shape, 1)
    fd = z.shape[-1]
    cand = jnp.where(z == m, iota, fd)
    idx = jnp.min(cand, axis=-1, keepdims=True)
    o_ref[...] = (iota == idx).astype(jnp.float32)


def kernel(logits, u):
    b, s, fd = logits.shape
    rows = b * s
    l2 = logits.reshape(rows, fd)
    u2 = u.reshape(rows, fd)
    out = pl.pallas_call(
        _mask_kernel,
        grid=(rows // _BLOCK_ROWS,),
        in_specs=[
            pl.BlockSpec((_BLOCK_ROWS, fd), lambda i: (i, 0)),
            pl.BlockSpec((_BLOCK_ROWS, fd), lambda i: (i, 0)),
        ],
        out_specs=pl.BlockSpec((_BLOCK_ROWS, fd), lambda i: (i, 0)),
        out_shape=jax.ShapeDtypeStruct((rows, fd), jnp.float32),
        compiler_params=pltpu.CompilerParams(
            dimension_semantics=("parallel",),
        ),
    )(l2, u2)
    return out.reshape(b, s, fd)


# speedup vs baseline: 427.0466x; 1.0505x over previous
import jax
import jax.numpy as jnp
from jax.experimental import pallas as pl
from jax.experimental.pallas import tpu as pltpu


_BLOCK_ROWS = 1024


def _mask_kernel(l_ref, u_ref, o_ref):
    z = l_ref[...] - jnp.log(-jnp.log(u_ref[...]))
    m = jnp.max(z, axis=-1, keepdims=True)
    iota = jax.lax.broadcasted_iota(jnp.int32, z.shape, 1)
    fd = z.shape[-1]
    cand = jnp.where(z == m, iota, fd)
    idx = jnp.min(cand, axis=-1, keepdims=True)
    o_ref[...] = (iota == idx).astype(jnp.float32)


def kernel(logits, u):
    b, s, fd = logits.shape
    rows = b * s
    l2 = logits.reshape(rows, fd)
    u2 = u.reshape(rows, fd)
    out = pl.pallas_call(
        _mask_kernel,
        grid=(rows // _BLOCK_ROWS,),
        in_specs=[
            pl.BlockSpec((_BLOCK_ROWS, fd), lambda i: (i, 0)),
            pl.BlockSpec((_BLOCK_ROWS, fd), lambda i: (i, 0)),
        ],
        out_specs=pl.BlockSpec((_BLOCK_ROWS, fd), lambda i: (i, 0)),
        out_shape=jax.ShapeDtypeStruct((rows, fd), jnp.float32),
        compiler_params=pltpu.CompilerParams(
            dimension_semantics=("parallel",),
        ),
    )(l2, u2)
    return out.reshape(b, s, fd)


# speedup vs baseline: 444.5713x; 1.0410x over previous
import jax
import jax.numpy as jnp
from jax.experimental import pallas as pl
from jax.experimental.pallas import tpu as pltpu


_BLOCK_ROWS = 2048


def _mask_kernel(l_ref, u_ref, o_ref):
    z = l_ref[...] - jnp.log(-jnp.log(u_ref[...]))
    m = jnp.max(z, axis=-1, keepdims=True)
    iota = jax.lax.broadcasted_iota(jnp.int32, z.shape, 1)
    fd = z.shape[-1]
    cand = jnp.where(z == m, iota, fd)
    idx = jnp.min(cand, axis=-1, keepdims=True)
    o_ref[...] = (iota == idx).astype(jnp.float32)


def kernel(logits, u):
    b, s, fd = logits.shape
    rows = b * s
    l2 = logits.reshape(rows, fd)
    u2 = u.reshape(rows, fd)
    out = pl.pallas_call(
        _mask_kernel,
        grid=(rows // _BLOCK_ROWS,),
        in_specs=[
            pl.BlockSpec((_BLOCK_ROWS, fd), lambda i: (i, 0)),
            pl.BlockSpec((_BLOCK_ROWS, fd), lambda i: (i, 0)),
        ],
        out_specs=pl.BlockSpec((_BLOCK_ROWS, fd), lambda i: (i, 0)),
        out_shape=jax.ShapeDtypeStruct((rows, fd), jnp.float32),
        compiler_params=pltpu.CompilerParams(
            dimension_semantics=("parallel",),
        ),
    )(l2, u2)
    return out.reshape(b, s, fd)
